# baseline (speedup 1.0000x reference)
import functools

import jax
import jax.numpy as jnp
from jax import lax
from jax.experimental import pallas as pl
from jax.experimental.pallas import tpu as pltpu
from jax.experimental.pallas import tpu_sc as plsc

EPS = 1e-06

_B, _S = 4096, 2048
_BTC = 2560
_BSC = _B - _BTC
_BB = 128
_NBT = _BTC // _BB

_NC, _NS, _L = 2, 16, 16
_NW = _NC * _NS
_RW = _BSC // _NW
_R = 4
_NCHUNK = _RW // _R
_NG = _S // 64


def _tc_body(yp_ref, lab_ref, ctl_ref, mf_ref, mc_ref, out_ref, acc_ref):
    i = pl.program_id(0)

    @pl.when(i == 0)
    def _init():
        acc_ref[...] = jnp.zeros_like(acc_ref)

    p0 = yp_ref[:, 0, :]
    p1 = yp_ref[:, 1, :]
    t = lab_ref[...]
    tc = ctl_ref[...]
    mf = mf_ref[...]
    mc = mc_ref[...]
    md = mf & mc

    full = p0 + p1
    diff = t - tc

    def fold(x):
        return jnp.sum(x.reshape(_BB // 8, 8, _S), axis=0)

    def sums(p, t_, m, base):
        u = jnp.where(m, p, 0.0)
        v = jnp.where(m, t_, 0.0)
        acc_ref[base + 0] += fold(jnp.where(m, 1.0, 0.0))
        acc_ref[base + 1] += fold(u)
        acc_ref[base + 2] += fold(v)
        acc_ref[base + 3] += fold(u * v)
        acc_ref[base + 4] += fold(u * u)
        acc_ref[base + 5] += fold(v * v)
        return u, v

    sums(p0, tc, mc, 0)
    u2, v2 = sums(full, t, mf, 6)
    acc_ref[12] += fold(jnp.abs(u2 - v2))
    sums(p1, diff, md, 13)

    @pl.when(i == _NBT - 1)
    def _fold_out():
        out_ref[...] = jnp.sum(
            acc_ref[...].reshape(19, 8, _S // 128, 128), axis=2)


def _tc_partials(y_pred, labels, labels_ctrl, mask_full, mask_ctrl):
    return pl.pallas_call(
        _tc_body,
        grid=(_NBT,),
        in_specs=[
            pl.BlockSpec((_BB, 2, _S), lambda i: (i, 0, 0)),
            pl.BlockSpec((_BB, _S), lambda i: (i, 0)),
            pl.BlockSpec((_BB, _S), lambda i: (i, 0)),
            pl.BlockSpec((_BB, _S), lambda i: (i, 0)),
            pl.BlockSpec((_BB, _S), lambda i: (i, 0)),
        ],
        out_specs=pl.BlockSpec((19, 8, 128), lambda i: (0, 0, 0)),
        out_shape=jax.ShapeDtypeStruct((19, 8, 128), jnp.float32),
        scratch_shapes=[pltpu.VMEM((19, 8, _S), jnp.float32)],
    )(y_pred, labels, labels_ctrl, mask_full, mask_ctrl)


def _sc_body(yp_hbm, lab_hbm, ctl_hbm, mf_hbm, mc_hbm, out_hbm,
             ypv, labv, ctlv, mfv, mcv, accv, sems):
    wid = lax.axis_index("s") * _NC + lax.axis_index("c")
    base = _BTC + wid * _RW
    mbase = wid * _RW

    def make_inner(par, r):
        def inner(g, accs):
            (n1, sp1, st1, spt1, spp1, stt1,
             n2, sp2, st2, spt2, spp2, stt2, sabs,
             n3, sp3, st3, spt3, spp3, stt3) = accs
            b64 = g * 64
            for j in range(4):
                c0 = b64 + 16 * j
                mf = mfv[par, r, pl.ds(c0, 16)]
                mc = mcv[par, r, pl.ds(c0, 16)]
                md = mf * mc
                p0 = ypv[par, r, 0, pl.ds(c0, 16)]
                p1 = ypv[par, r, 1, pl.ds(c0, 16)]
                t = labv[par, r, pl.ds(c0, 16)]
                tc = ctlv[par, r, pl.ds(c0, 16)]
                full = p0 + p1
                diff = t - tc
                u1 = p0 * mc
                v1 = tc * mc
                u2 = full * mf
                v2 = t * mf
                u3 = p1 * md
                v3 = diff * md
                n1 += mc
                sp1 += u1
                st1 += v1
                spt1 += u1 * v1
                spp1 += u1 * u1
                stt1 += v1 * v1
                n2 += mf
                sp2 += u2
                st2 += v2
                spt2 += u2 * v2
                spp2 += u2 * u2
                stt2 += v2 * v2
                sabs += jnp.abs(u2 - v2)
                n3 += md
                sp3 += u3
                st3 += v3
                spt3 += u3 * v3
                spp3 += u3 * u3
                stt3 += v3 * v3
            return (n1, sp1, st1, spt1, spp1, stt1,
                    n2, sp2, st2, spt2, spp2, stt2, sabs,
                    n3, sp3, st3, spt3, spp3, stt3)
        return inner

    def issue(ch, par):
        row = base + ch * _R
        mrow = mbase + ch * _R
        pltpu.async_copy(yp_hbm.at[pl.ds(row, _R)], ypv.at[par], sems.at[par])
        pltpu.async_copy(lab_hbm.at[pl.ds(row, _R)], labv.at[par], sems.at[par])
        pltpu.async_copy(ctl_hbm.at[pl.ds(row, _R)], ctlv.at[par], sems.at[par])
        pltpu.async_copy(mf_hbm.at[pl.ds(mrow, _R)], mfv.at[par], sems.at[par])
        pltpu.async_copy(mc_hbm.at[pl.ds(mrow, _R)], mcv.at[par], sems.at[par])

    def wait(par):
        row = base
        pltpu.make_async_copy(yp_hbm.at[pl.ds(row, _R)], ypv.at[par], sems.at[par]).wait()
        pltpu.make_async_copy(lab_hbm.at[pl.ds(row, _R)], labv.at[par], sems.at[par]).wait()
        pltpu.make_async_copy(ctl_hbm.at[pl.ds(row, _R)], ctlv.at[par], sems.at[par]).wait()
        pltpu.make_async_copy(mf_hbm.at[pl.ds(row, _R)], mfv.at[par], sems.at[par]).wait()
        pltpu.make_async_copy(mc_hbm.at[pl.ds(row, _R)], mcv.at[par], sems.at[par]).wait()

    def compute(par, accs):
        for r in range(_R):
            accs = lax.fori_loop(0, _NG, make_inner(par, r), accs, unroll=2)
        return accs

    def pair_step(k, accs):
        ch0 = 2 * k
        wait(0)
        accs = compute(0, accs)

        @pl.when(ch0 + 2 < _NCHUNK)
        def _():
            issue(ch0 + 2, 0)

        wait(1)
        accs = compute(1, accs)

        @pl.when(ch0 + 3 < _NCHUNK)
        def _():
            issue(ch0 + 3, 1)

        return accs

    issue(0, 0)
    issue(1, 1)
    zero = jnp.zeros((16,), jnp.float32)
    accs = lax.fori_loop(0, _NCHUNK // 2, pair_step, (zero,) * 19)
    for k in range(19):
        accv[k, :] = accs[k]
    pltpu.sync_copy(accv, out_hbm.at[wid])


@functools.partial(
    pl.kernel,
    out_type=jax.ShapeDtypeStruct((_NW, 19, 16), jnp.float32),
    mesh=plsc.VectorSubcoreMesh(core_axis_name="c", subcore_axis_name="s"),
    scratch_types=[
        pltpu.VMEM((2, _R, 2, _S), jnp.float32),
        pltpu.VMEM((2, _R, _S), jnp.float32),
        pltpu.VMEM((2, _R, _S), jnp.float32),
        pltpu.VMEM((2, _R, _S), jnp.float32),
        pltpu.VMEM((2, _R, _S), jnp.float32),
        pltpu.VMEM((19, 16), jnp.float32),
        pltpu.SemaphoreType.DMA((2,)),
    ],
)
def _sc_partials(*args):
    _sc_body(*args)


def _fin_body(tc_ref, sc_ref, out_ref):
    s = (jnp.sum(tc_ref[...], axis=(1, 2)) +
         jnp.sum(sc_ref[...], axis=(0, 2)))

    def corr(base):
        n = s[base + 0]
        sp = s[base + 1]
        st = s[base + 2]
        spt = s[base + 3]
        spp = s[base + 4]
        stt = s[base + 5]
        dot = spt - sp * st / n
        na = jnp.sqrt(spp - sp * sp / n)
        nb = jnp.sqrt(stt - st * st / n)
        return dot / (jnp.maximum(na, EPS) * jnp.maximum(nb, EPS)), n

    corr_ctrl, _ = corr(0)
    corr_full, n2 = corr(6)
    corr_diff, _ = corr(13)
    l1 = jnp.sqrt(s[12] / n2)
    out_ref[0] = 1.0 - corr_ctrl
    out_ref[1] = (1.0 - corr_full) + l1
    out_ref[2] = corr_full
    out_ref[3] = l1
    out_ref[4] = 1.0 - corr_diff


@jax.jit
def _reduce(y_pred, labels, labels_ctrl, mask_full, mask_ctrl):
    sc_part = _sc_partials(y_pred, labels, labels_ctrl,
                           mask_full[_BTC:].astype(jnp.float32),
                           mask_ctrl[_BTC:].astype(jnp.float32))
    tc_part = _tc_partials(y_pred, labels, labels_ctrl, mask_full, mask_ctrl)
    return pl.pallas_call(
        _fin_body,
        out_specs=pl.BlockSpec(memory_space=pltpu.SMEM),
        out_shape=jax.ShapeDtypeStruct((8,), jnp.float32),
    )(tc_part, sc_part)


def kernel(y_pred, labels, labels_ctrl, mask_full, mask_ctrl, condition_):
    out = _reduce(y_pred, labels, labels_ctrl, mask_full, mask_ctrl)
    loss_ctrl, loss_full, perf, l1, loss_depr = (
        out[0], out[1], out[2], out[3], out[4])
    loss = jnp.where(condition_ != 64,
                     loss_ctrl + loss_depr + loss_full,
                     loss_ctrl + loss_full)
    return (loss, perf, l1)


# speedup vs baseline: 1.0139x; 1.0139x over previous
import functools

import jax
import jax.numpy as jnp
from jax import lax
from jax.experimental import pallas as pl
from jax.experimental.pallas import tpu as pltpu
from jax.experimental.pallas import tpu_sc as plsc

EPS = 1e-06

_B, _S = 4096, 2048
_BTC = 2816
_BSC = _B - _BTC
_BB = 128
_NBT = _BTC // _BB

_NC, _NS, _L = 2, 16, 16
_NW = _NC * _NS
_RW = _BSC // _NW
_R = 4
_NCHUNK = _RW // _R
_NG = _S // 64


def _tc_body(yp_ref, lab_ref, ctl_ref, mf_ref, mc_ref, out_ref, acc_ref):
    i = pl.program_id(0)

    @pl.when(i == 0)
    def _init():
        acc_ref[...] = jnp.zeros_like(acc_ref)

    p0 = yp_ref[:, 0, :]
    p1 = yp_ref[:, 1, :]
    t = lab_ref[...]
    tc = ctl_ref[...]
    mf = mf_ref[...]
    mc = mc_ref[...]
    md = mf & mc

    full = p0 + p1
    diff = t - tc

    def fold(x):
        return jnp.sum(x.reshape(_BB // 8, 8, _S), axis=0)

    def sums(p, t_, m, base):
        u = jnp.where(m, p, 0.0)
        v = jnp.where(m, t_, 0.0)
        acc_ref[base + 0] += fold(jnp.where(m, 1.0, 0.0))
        acc_ref[base + 1] += fold(u)
        acc_ref[base + 2] += fold(v)
        acc_ref[base + 3] += fold(u * v)
        acc_ref[base + 4] += fold(u * u)
        acc_ref[base + 5] += fold(v * v)
        return u, v

    sums(p0, tc, mc, 0)
    u2, v2 = sums(full, t, mf, 6)
    acc_ref[12] += fold(jnp.abs(u2 - v2))
    sums(p1, diff, md, 13)

    @pl.when(i == _NBT - 1)
    def _fold_out():
        out_ref[...] = jnp.sum(
            acc_ref[...].reshape(19, 8, _S // 128, 128), axis=2)


def _tc_partials(y_pred, labels, labels_ctrl, mask_full, mask_ctrl):
    return pl.pallas_call(
        _tc_body,
        grid=(_NBT,),
        in_specs=[
            pl.BlockSpec((_BB, 2, _S), lambda i: (i, 0, 0)),
            pl.BlockSpec((_BB, _S), lambda i: (i, 0)),
            pl.BlockSpec((_BB, _S), lambda i: (i, 0)),
            pl.BlockSpec((_BB, _S), lambda i: (i, 0)),
            pl.BlockSpec((_BB, _S), lambda i: (i, 0)),
        ],
        out_specs=pl.BlockSpec((19, 8, 128), lambda i: (0, 0, 0)),
        out_shape=jax.ShapeDtypeStruct((19, 8, 128), jnp.float32),
        scratch_shapes=[pltpu.VMEM((19, 8, _S), jnp.float32)],
    )(y_pred, labels, labels_ctrl, mask_full, mask_ctrl)


def _sc_body(yp_hbm, lab_hbm, ctl_hbm, mf_hbm, mc_hbm, out_hbm,
             ypv, labv, ctlv, mfv, mcv, accv, sems):
    wid = lax.axis_index("s") * _NC + lax.axis_index("c")
    base = _BTC + wid * _RW
    mbase = wid * _RW

    def make_inner(par, r):
        def inner(g, accs):
            (n1, sp1, st1, spt1, spp1, stt1,
             n2, sp2, st2, spt2, spp2, stt2, sabs,
             n3, sp3, st3, spt3, spp3, stt3) = accs
            b64 = g * 64
            for j in range(4):
                c0 = b64 + 16 * j
                mf = mfv[par, r, pl.ds(c0, 16)]
                mc = mcv[par, r, pl.ds(c0, 16)]
                md = mf * mc
                p0 = ypv[par, r, 0, pl.ds(c0, 16)]
                p1 = ypv[par, r, 1, pl.ds(c0, 16)]
                t = labv[par, r, pl.ds(c0, 16)]
                tc = ctlv[par, r, pl.ds(c0, 16)]
                full = p0 + p1
                diff = t - tc
                u1 = p0 * mc
                v1 = tc * mc
                u2 = full * mf
                v2 = t * mf
                u3 = p1 * md
                v3 = diff * md
                n1 += mc
                sp1 += u1
                st1 += v1
                spt1 += u1 * v1
                spp1 += u1 * u1
                stt1 += v1 * v1
                n2 += mf
                sp2 += u2
                st2 += v2
                spt2 += u2 * v2
                spp2 += u2 * u2
                stt2 += v2 * v2
                sabs += jnp.abs(u2 - v2)
                n3 += md
                sp3 += u3
                st3 += v3
                spt3 += u3 * v3
                spp3 += u3 * u3
                stt3 += v3 * v3
            return (n1, sp1, st1, spt1, spp1, stt1,
                    n2, sp2, st2, spt2, spp2, stt2, sabs,
                    n3, sp3, st3, spt3, spp3, stt3)
        return inner

    def issue(ch, par):
        row = base + ch * _R
        mrow = mbase + ch * _R
        pltpu.async_copy(yp_hbm.at[pl.ds(row, _R)], ypv.at[par], sems.at[par])
        pltpu.async_copy(lab_hbm.at[pl.ds(row, _R)], labv.at[par], sems.at[par])
        pltpu.async_copy(ctl_hbm.at[pl.ds(row, _R)], ctlv.at[par], sems.at[par])
        pltpu.async_copy(mf_hbm.at[pl.ds(mrow, _R)], mfv.at[par], sems.at[par])
        pltpu.async_copy(mc_hbm.at[pl.ds(mrow, _R)], mcv.at[par], sems.at[par])

    def wait(par):
        row = base
        pltpu.make_async_copy(yp_hbm.at[pl.ds(row, _R)], ypv.at[par], sems.at[par]).wait()
        pltpu.make_async_copy(lab_hbm.at[pl.ds(row, _R)], labv.at[par], sems.at[par]).wait()
        pltpu.make_async_copy(ctl_hbm.at[pl.ds(row, _R)], ctlv.at[par], sems.at[par]).wait()
        pltpu.make_async_copy(mf_hbm.at[pl.ds(row, _R)], mfv.at[par], sems.at[par]).wait()
        pltpu.make_async_copy(mc_hbm.at[pl.ds(row, _R)], mcv.at[par], sems.at[par]).wait()

    def compute(par, accs):
        for r in range(_R):
            accs = lax.fori_loop(0, _NG, make_inner(par, r), accs)
        return accs

    def pair_step(k, accs):
        ch0 = 2 * k
        wait(0)
        accs = compute(0, accs)

        @pl.when(ch0 + 2 < _NCHUNK)
        def _():
            issue(ch0 + 2, 0)

        wait(1)
        accs = compute(1, accs)

        @pl.when(ch0 + 3 < _NCHUNK)
        def _():
            issue(ch0 + 3, 1)

        return accs

    issue(0, 0)
    issue(1, 1)
    zero = jnp.zeros((16,), jnp.float32)
    accs = lax.fori_loop(0, _NCHUNK // 2, pair_step, (zero,) * 19)
    for k in range(19):
        accv[k, :] = accs[k]
    pltpu.sync_copy(accv, out_hbm.at[wid])


@functools.partial(
    pl.kernel,
    out_type=jax.ShapeDtypeStruct((_NW, 19, 16), jnp.float32),
    mesh=plsc.VectorSubcoreMesh(core_axis_name="c", subcore_axis_name="s"),
    scratch_types=[
        pltpu.VMEM((2, _R, 2, _S), jnp.float32),
        pltpu.VMEM((2, _R, _S), jnp.float32),
        pltpu.VMEM((2, _R, _S), jnp.float32),
        pltpu.VMEM((2, _R, _S), jnp.float32),
        pltpu.VMEM((2, _R, _S), jnp.float32),
        pltpu.VMEM((19, 16), jnp.float32),
        pltpu.SemaphoreType.DMA((2,)),
    ],
)
def _sc_partials(*args):
    _sc_body(*args)


def _fin_body(tc_ref, sc_ref, out_ref):
    s = (jnp.sum(tc_ref[...], axis=(1, 2)) +
         jnp.sum(sc_ref[...], axis=(0, 2)))

    def corr(base):
        n = s[base + 0]
        sp = s[base + 1]
        st = s[base + 2]
        spt = s[base + 3]
        spp = s[base + 4]
        stt = s[base + 5]
        dot = spt - sp * st / n
        na = jnp.sqrt(spp - sp * sp / n)
        nb = jnp.sqrt(stt - st * st / n)
        return dot / (jnp.maximum(na, EPS) * jnp.maximum(nb, EPS)), n

    corr_ctrl, _ = corr(0)
    corr_full, n2 = corr(6)
    corr_diff, _ = corr(13)
    l1 = jnp.sqrt(s[12] / n2)
    out_ref[0] = 1.0 - corr_ctrl
    out_ref[1] = (1.0 - corr_full) + l1
    out_ref[2] = corr_full
    out_ref[3] = l1
    out_ref[4] = 1.0 - corr_diff


@jax.jit
def _reduce(y_pred, labels, labels_ctrl, mask_full, mask_ctrl):
    sc_part = _sc_partials(y_pred, labels, labels_ctrl,
                           mask_full[_BTC:].astype(jnp.float32),
                           mask_ctrl[_BTC:].astype(jnp.float32))
    tc_part = _tc_partials(y_pred, labels, labels_ctrl, mask_full, mask_ctrl)
    return pl.pallas_call(
        _fin_body,
        out_specs=pl.BlockSpec(memory_space=pltpu.SMEM),
        out_shape=jax.ShapeDtypeStruct((8,), jnp.float32),
    )(tc_part, sc_part)


def kernel(y_pred, labels, labels_ctrl, mask_full, mask_ctrl, condition_):
    out = _reduce(y_pred, labels, labels_ctrl, mask_full, mask_ctrl)
    loss_ctrl, loss_full, perf, l1, loss_depr = (
        out[0], out[1], out[2], out[3], out[4])
    loss = jnp.where(condition_ != 64,
                     loss_ctrl + loss_depr + loss_full,
                     loss_ctrl + loss_full)
    return (loss, perf, l1)


# speedup vs baseline: 1.0736x; 1.0589x over previous
import functools

import jax
import jax.numpy as jnp
from jax import lax
from jax.experimental import pallas as pl
from jax.experimental.pallas import tpu as pltpu
from jax.experimental.pallas import tpu_sc as plsc

EPS = 1e-06

_B, _S = 4096, 2048
_BTC = 2304
_BSC = _B - _BTC
_BB = 128
_NBT = _BTC // _BB

_NC, _NS, _L = 2, 16, 16
_NW = _NC * _NS
_RW = _BSC // _NW
_R = 4
_NCHUNK = _RW // _R
_NG = _S // 64


def _tc_body(yp_ref, lab_ref, ctl_ref, mf_ref, mc_ref, out_ref, acc_ref):
    i = pl.program_id(0)

    @pl.when(i == 0)
    def _init():
        acc_ref[...] = jnp.zeros_like(acc_ref)

    p0 = yp_ref[:, 0, :]
    p1 = yp_ref[:, 1, :]
    t = lab_ref[...]
    tc = ctl_ref[...]
    mf = mf_ref[...]
    mc = mc_ref[...]
    md = mf & mc

    full = p0 + p1
    diff = t - tc

    def fold(x):
        return jnp.sum(x.reshape(_BB // 8, 8, _S), axis=0)

    def sums(p, t_, m, base):
        u = jnp.where(m, p, 0.0)
        v = jnp.where(m, t_, 0.0)
        acc_ref[base + 0] += fold(jnp.where(m, 1.0, 0.0))
        acc_ref[base + 1] += fold(u)
        acc_ref[base + 2] += fold(v)
        acc_ref[base + 3] += fold(u * v)
        acc_ref[base + 4] += fold(u * u)
        acc_ref[base + 5] += fold(v * v)
        return u, v

    sums(p0, tc, mc, 0)
    u2, v2 = sums(full, t, mf, 6)
    acc_ref[12] += fold(jnp.abs(u2 - v2))
    sums(p1, diff, md, 13)

    @pl.when(i == _NBT - 1)
    def _fold_out():
        out_ref[...] = jnp.sum(
            acc_ref[...].reshape(19, 8, _S // 128, 128), axis=2)


def _tc_partials(y_pred, labels, labels_ctrl, mask_full, mask_ctrl):
    return pl.pallas_call(
        _tc_body,
        grid=(_NBT,),
        in_specs=[
            pl.BlockSpec((_BB, 2, _S), lambda i: (i, 0, 0)),
            pl.BlockSpec((_BB, _S), lambda i: (i, 0)),
            pl.BlockSpec((_BB, _S), lambda i: (i, 0)),
            pl.BlockSpec((_BB, _S), lambda i: (i, 0)),
            pl.BlockSpec((_BB, _S), lambda i: (i, 0)),
        ],
        out_specs=pl.BlockSpec((19, 8, 128), lambda i: (0, 0, 0)),
        out_shape=jax.ShapeDtypeStruct((19, 8, 128), jnp.float32),
        scratch_shapes=[pltpu.VMEM((19, 8, _S), jnp.float32)],
    )(y_pred, labels, labels_ctrl, mask_full, mask_ctrl)


def _sc_body(yp_hbm, lab_hbm, ctl_hbm, mf_hbm, mc_hbm, out_hbm,
             ypv, labv, ctlv, mfv, mcv, accv, sems):
    wid = lax.axis_index("s") * _NC + lax.axis_index("c")
    base = _BTC + wid * _RW
    mbase = wid * _RW

    def make_inner(par, r):
        def inner(g, accs):
            (n1, sp1, st1, spt1, spp1, stt1,
             n2, sp2, st2, spt2, spp2, stt2, sabs,
             n3, sp3, st3, spt3, spp3, stt3) = accs
            b64 = g * 64
            for j in range(4):
                c0 = b64 + 16 * j
                mf = mfv[par, r, pl.ds(c0, 16)]
                mc = mcv[par, r, pl.ds(c0, 16)]
                md = mf * mc
                p0 = ypv[par, r, 0, pl.ds(c0, 16)]
                p1 = ypv[par, r, 1, pl.ds(c0, 16)]
                t = labv[par, r, pl.ds(c0, 16)]
                tc = ctlv[par, r, pl.ds(c0, 16)]
                full = p0 + p1
                diff = t - tc
                u1 = p0 * mc
                v1 = tc * mc
                u2 = full * mf
                v2 = t * mf
                u3 = p1 * md
                v3 = diff * md
                n1 += mc
                sp1 += u1
                st1 += v1
                spt1 += u1 * v1
                spp1 += u1 * u1
                stt1 += v1 * v1
                n2 += mf
                sp2 += u2
                st2 += v2
                spt2 += u2 * v2
                spp2 += u2 * u2
                stt2 += v2 * v2
                sabs += jnp.abs(u2 - v2)
                n3 += md
                sp3 += u3
                st3 += v3
                spt3 += u3 * v3
                spp3 += u3 * u3
                stt3 += v3 * v3
            return (n1, sp1, st1, spt1, spp1, stt1,
                    n2, sp2, st2, spt2, spp2, stt2, sabs,
                    n3, sp3, st3, spt3, spp3, stt3)
        return inner

    def issue(ch, par):
        row = base + ch * _R
        mrow = mbase + ch * _R
        pltpu.async_copy(yp_hbm.at[pl.ds(row, _R)], ypv.at[par], sems.at[par])
        pltpu.async_copy(lab_hbm.at[pl.ds(row, _R)], labv.at[par], sems.at[par])
        pltpu.async_copy(ctl_hbm.at[pl.ds(row, _R)], ctlv.at[par], sems.at[par])
        pltpu.async_copy(mf_hbm.at[pl.ds(mrow, _R)], mfv.at[par], sems.at[par])
        pltpu.async_copy(mc_hbm.at[pl.ds(mrow, _R)], mcv.at[par], sems.at[par])

    def wait(par):
        row = base
        pltpu.make_async_copy(yp_hbm.at[pl.ds(row, _R)], ypv.at[par], sems.at[par]).wait()
        pltpu.make_async_copy(lab_hbm.at[pl.ds(row, _R)], labv.at[par], sems.at[par]).wait()
        pltpu.make_async_copy(ctl_hbm.at[pl.ds(row, _R)], ctlv.at[par], sems.at[par]).wait()
        pltpu.make_async_copy(mf_hbm.at[pl.ds(row, _R)], mfv.at[par], sems.at[par]).wait()
        pltpu.make_async_copy(mc_hbm.at[pl.ds(row, _R)], mcv.at[par], sems.at[par]).wait()

    def compute(par, accs):
        for r in range(_R):
            accs = lax.fori_loop(0, _NG, make_inner(par, r), accs)
        return accs

    def pair_step(k, accs):
        ch0 = 2 * k
        wait(0)
        accs = compute(0, accs)

        @pl.when(ch0 + 2 < _NCHUNK)
        def _():
            issue(ch0 + 2, 0)

        wait(1)
        accs = compute(1, accs)

        @pl.when(ch0 + 3 < _NCHUNK)
        def _():
            issue(ch0 + 3, 1)

        return accs

    issue(0, 0)
    issue(1, 1)
    zero = jnp.zeros((16,), jnp.float32)
    accs = lax.fori_loop(0, _NCHUNK // 2, pair_step, (zero,) * 19)
    for k in range(19):
        accv[k, :] = accs[k]
    pltpu.sync_copy(accv, out_hbm.at[wid])


@functools.partial(
    pl.kernel,
    out_type=jax.ShapeDtypeStruct((_NW, 19, 16), jnp.float32),
    mesh=plsc.VectorSubcoreMesh(core_axis_name="c", subcore_axis_name="s"),
    scratch_types=[
        pltpu.VMEM((2, _R, 2, _S), jnp.float32),
        pltpu.VMEM((2, _R, _S), jnp.float32),
        pltpu.VMEM((2, _R, _S), jnp.float32),
        pltpu.VMEM((2, _R, _S), jnp.float32),
        pltpu.VMEM((2, _R, _S), jnp.float32),
        pltpu.VMEM((19, 16), jnp.float32),
        pltpu.SemaphoreType.DMA((2,)),
    ],
)
def _sc_partials(*args):
    _sc_body(*args)


def _fin_body(tc_ref, sc_ref, out_ref):
    s = (jnp.sum(tc_ref[...], axis=(1, 2)) +
         jnp.sum(sc_ref[...], axis=(0, 2)))

    def corr(base):
        n = s[base + 0]
        sp = s[base + 1]
        st = s[base + 2]
        spt = s[base + 3]
        spp = s[base + 4]
        stt = s[base + 5]
        dot = spt - sp * st / n
        na = jnp.sqrt(spp - sp * sp / n)
        nb = jnp.sqrt(stt - st * st / n)
        return dot / (jnp.maximum(na, EPS) * jnp.maximum(nb, EPS)), n

    corr_ctrl, _ = corr(0)
    corr_full, n2 = corr(6)
    corr_diff, _ = corr(13)
    l1 = jnp.sqrt(s[12] / n2)
    out_ref[0] = 1.0 - corr_ctrl
    out_ref[1] = (1.0 - corr_full) + l1
    out_ref[2] = corr_full
    out_ref[3] = l1
    out_ref[4] = 1.0 - corr_diff


@jax.jit
def _reduce(y_pred, labels, labels_ctrl, mask_full, mask_ctrl):
    sc_part = _sc_partials(y_pred, labels, labels_ctrl,
                           mask_full[_BTC:].astype(jnp.float32),
                           mask_ctrl[_BTC:].astype(jnp.float32))
    tc_part = _tc_partials(y_pred, labels, labels_ctrl, mask_full, mask_ctrl)
    return pl.pallas_call(
        _fin_body,
        out_specs=pl.BlockSpec(memory_space=pltpu.SMEM),
        out_shape=jax.ShapeDtypeStruct((8,), jnp.float32),
    )(tc_part, sc_part)


def kernel(y_pred, labels, labels_ctrl, mask_full, mask_ctrl, condition_):
    out = _reduce(y_pred, labels, labels_ctrl, mask_full, mask_ctrl)
    loss_ctrl, loss_full, perf, l1, loss_depr = (
        out[0], out[1], out[2], out[3], out[4])
    loss = jnp.where(condition_ != 64,
                     loss_ctrl + loss_depr + loss_full,
                     loss_ctrl + loss_full)
    return (loss, perf, l1)


# speedup vs baseline: 1.0976x; 1.0224x over previous
import functools

import jax
import jax.numpy as jnp
from jax import lax
from jax.experimental import pallas as pl
from jax.experimental.pallas import tpu as pltpu
from jax.experimental.pallas import tpu_sc as plsc

EPS = 1e-06

_B, _S = 4096, 2048
_BTC = 2048
_BSC = _B - _BTC
_BB = 128
_NBT = _BTC // _BB

_NC, _NS, _L = 2, 16, 16
_NW = _NC * _NS
_RW = _BSC // _NW
_R = 4
_NCHUNK = _RW // _R
_NG = _S // 64


def _tc_body(yp_ref, lab_ref, ctl_ref, mf_ref, mc_ref, out_ref, acc_ref):
    i = pl.program_id(0)

    @pl.when(i == 0)
    def _init():
        acc_ref[...] = jnp.zeros_like(acc_ref)

    p0 = yp_ref[:, 0, :]
    p1 = yp_ref[:, 1, :]
    t = lab_ref[...]
    tc = ctl_ref[...]
    mf = mf_ref[...]
    mc = mc_ref[...]
    md = mf & mc

    full = p0 + p1
    diff = t - tc

    def fold(x):
        return jnp.sum(x.reshape(_BB // 8, 8, _S), axis=0)

    def sums(p, t_, m, base):
        u = jnp.where(m, p, 0.0)
        v = jnp.where(m, t_, 0.0)
        acc_ref[base + 0] += fold(jnp.where(m, 1.0, 0.0))
        acc_ref[base + 1] += fold(u)
        acc_ref[base + 2] += fold(v)
        acc_ref[base + 3] += fold(u * v)
        acc_ref[base + 4] += fold(u * u)
        acc_ref[base + 5] += fold(v * v)
        return u, v

    sums(p0, tc, mc, 0)
    u2, v2 = sums(full, t, mf, 6)
    acc_ref[12] += fold(jnp.abs(u2 - v2))
    sums(p1, diff, md, 13)

    @pl.when(i == _NBT - 1)
    def _fold_out():
        out_ref[...] = jnp.sum(
            acc_ref[...].reshape(19, 8, _S // 128, 128), axis=2)


def _tc_partials(y_pred, labels, labels_ctrl, mask_full, mask_ctrl):
    return pl.pallas_call(
        _tc_body,
        grid=(_NBT,),
        in_specs=[
            pl.BlockSpec((_BB, 2, _S), lambda i: (i, 0, 0)),
            pl.BlockSpec((_BB, _S), lambda i: (i, 0)),
            pl.BlockSpec((_BB, _S), lambda i: (i, 0)),
            pl.BlockSpec((_BB, _S), lambda i: (i, 0)),
            pl.BlockSpec((_BB, _S), lambda i: (i, 0)),
        ],
        out_specs=pl.BlockSpec((19, 8, 128), lambda i: (0, 0, 0)),
        out_shape=jax.ShapeDtypeStruct((19, 8, 128), jnp.float32),
        scratch_shapes=[pltpu.VMEM((19, 8, _S), jnp.float32)],
    )(y_pred, labels, labels_ctrl, mask_full, mask_ctrl)


def _sc_body(yp_hbm, lab_hbm, ctl_hbm, mf_hbm, mc_hbm, out_hbm,
             ypv, labv, ctlv, mfv, mcv, accv, sems):
    wid = lax.axis_index("s") * _NC + lax.axis_index("c")
    base = _BTC + wid * _RW
    mbase = wid * _RW

    def make_inner(par, r):
        def inner(g, accs):
            (n1, sp1, st1, spt1, spp1, stt1,
             n2, sp2, st2, spt2, spp2, stt2, sabs,
             n3, sp3, st3, spt3, spp3, stt3) = accs
            b64 = g * 64
            for j in range(4):
                c0 = b64 + 16 * j
                mf = mfv[par, r, pl.ds(c0, 16)]
                mc = mcv[par, r, pl.ds(c0, 16)]
                md = mf * mc
                p0 = ypv[par, r, 0, pl.ds(c0, 16)]
                p1 = ypv[par, r, 1, pl.ds(c0, 16)]
                t = labv[par, r, pl.ds(c0, 16)]
                tc = ctlv[par, r, pl.ds(c0, 16)]
                full = p0 + p1
                diff = t - tc
                u1 = p0 * mc
                v1 = tc * mc
                u2 = full * mf
                v2 = t * mf
                u3 = p1 * md
                v3 = diff * md
                n1 += mc
                sp1 += u1
                st1 += v1
                spt1 += u1 * v1
                spp1 += u1 * u1
                stt1 += v1 * v1
                n2 += mf
                sp2 += u2
                st2 += v2
                spt2 += u2 * v2
                spp2 += u2 * u2
                stt2 += v2 * v2
                sabs += jnp.abs(u2 - v2)
                n3 += md
                sp3 += u3
                st3 += v3
                spt3 += u3 * v3
                spp3 += u3 * u3
                stt3 += v3 * v3
            return (n1, sp1, st1, spt1, spp1, stt1,
                    n2, sp2, st2, spt2, spp2, stt2, sabs,
                    n3, sp3, st3, spt3, spp3, stt3)
        return inner

    def issue(ch, par):
        row = base + ch * _R
        mrow = mbase + ch * _R
        pltpu.async_copy(yp_hbm.at[pl.ds(row, _R)], ypv.at[par], sems.at[par])
        pltpu.async_copy(lab_hbm.at[pl.ds(row, _R)], labv.at[par], sems.at[par])
        pltpu.async_copy(ctl_hbm.at[pl.ds(row, _R)], ctlv.at[par], sems.at[par])
        pltpu.async_copy(mf_hbm.at[pl.ds(mrow, _R)], mfv.at[par], sems.at[par])
        pltpu.async_copy(mc_hbm.at[pl.ds(mrow, _R)], mcv.at[par], sems.at[par])

    def wait(par):
        row = base
        pltpu.make_async_copy(yp_hbm.at[pl.ds(row, _R)], ypv.at[par], sems.at[par]).wait()
        pltpu.make_async_copy(lab_hbm.at[pl.ds(row, _R)], labv.at[par], sems.at[par]).wait()
        pltpu.make_async_copy(ctl_hbm.at[pl.ds(row, _R)], ctlv.at[par], sems.at[par]).wait()
        pltpu.make_async_copy(mf_hbm.at[pl.ds(row, _R)], mfv.at[par], sems.at[par]).wait()
        pltpu.make_async_copy(mc_hbm.at[pl.ds(row, _R)], mcv.at[par], sems.at[par]).wait()

    def compute(par, accs):
        for r in range(_R):
            accs = lax.fori_loop(0, _NG, make_inner(par, r), accs)
        return accs

    def pair_step(k, accs):
        ch0 = 2 * k
        wait(0)
        accs = compute(0, accs)

        @pl.when(ch0 + 2 < _NCHUNK)
        def _():
            issue(ch0 + 2, 0)

        wait(1)
        accs = compute(1, accs)

        @pl.when(ch0 + 3 < _NCHUNK)
        def _():
            issue(ch0 + 3, 1)

        return accs

    issue(0, 0)
    issue(1, 1)
    zero = jnp.zeros((16,), jnp.float32)
    accs = lax.fori_loop(0, _NCHUNK // 2, pair_step, (zero,) * 19)
    for k in range(19):
        accv[k, :] = accs[k]
    pltpu.sync_copy(accv, out_hbm.at[wid])


@functools.partial(
    pl.kernel,
    out_type=jax.ShapeDtypeStruct((_NW, 19, 16), jnp.float32),
    mesh=plsc.VectorSubcoreMesh(core_axis_name="c", subcore_axis_name="s"),
    scratch_types=[
        pltpu.VMEM((2, _R, 2, _S), jnp.float32),
        pltpu.VMEM((2, _R, _S), jnp.float32),
        pltpu.VMEM((2, _R, _S), jnp.float32),
        pltpu.VMEM((2, _R, _S), jnp.float32),
        pltpu.VMEM((2, _R, _S), jnp.float32),
        pltpu.VMEM((19, 16), jnp.float32),
        pltpu.SemaphoreType.DMA((2,)),
    ],
)
def _sc_partials(*args):
    _sc_body(*args)


def _fin_body(tc_ref, sc_ref, out_ref):
    s = (jnp.sum(tc_ref[...], axis=(1, 2)) +
         jnp.sum(sc_ref[...], axis=(0, 2)))

    def corr(base):
        n = s[base + 0]
        sp = s[base + 1]
        st = s[base + 2]
        spt = s[base + 3]
        spp = s[base + 4]
        stt = s[base + 5]
        dot = spt - sp * st / n
        na = jnp.sqrt(spp - sp * sp / n)
        nb = jnp.sqrt(stt - st * st / n)
        return dot / (jnp.maximum(na, EPS) * jnp.maximum(nb, EPS)), n

    corr_ctrl, _ = corr(0)
    corr_full, n2 = corr(6)
    corr_diff, _ = corr(13)
    l1 = jnp.sqrt(s[12] / n2)
    out_ref[0] = 1.0 - corr_ctrl
    out_ref[1] = (1.0 - corr_full) + l1
    out_ref[2] = corr_full
    out_ref[3] = l1
    out_ref[4] = 1.0 - corr_diff


@jax.jit
def _reduce(y_pred, labels, labels_ctrl, mask_full, mask_ctrl):
    sc_part = _sc_partials(y_pred, labels, labels_ctrl,
                           mask_full[_BTC:].astype(jnp.float32),
                           mask_ctrl[_BTC:].astype(jnp.float32))
    tc_part = _tc_partials(y_pred, labels, labels_ctrl, mask_full, mask_ctrl)
    return pl.pallas_call(
        _fin_body,
        out_specs=pl.BlockSpec(memory_space=pltpu.SMEM),
        out_shape=jax.ShapeDtypeStruct((8,), jnp.float32),
    )(tc_part, sc_part)


def kernel(y_pred, labels, labels_ctrl, mask_full, mask_ctrl, condition_):
    out = _reduce(y_pred, labels, labels_ctrl, mask_full, mask_ctrl)
    loss_ctrl, loss_full, perf, l1, loss_depr = (
        out[0], out[1], out[2], out[3], out[4])
    loss = jnp.where(condition_ != 64,
                     loss_ctrl + loss_depr + loss_full,
                     loss_ctrl + loss_full)
    return (loss, perf, l1)
